# baseline (speedup 1.0000x reference)
import jax
import jax.numpy as jnp
from jax.experimental import pallas as pl
from jax.experimental.pallas import tpu as pltpu

_NC = 4
_CH = 196 // _NC


def _copy_body(in_ref, out_ref, buf, isems, osems):
    for i in range(_NC):
        pltpu.make_async_copy(
            in_ref.at[pl.ds(i * _CH, _CH)],
            buf.at[pl.ds(i * _CH, _CH)],
            isems.at[i],
        ).start()
    for i in range(_NC):
        pltpu.make_async_copy(
            in_ref.at[pl.ds(i * _CH, _CH)],
            buf.at[pl.ds(i * _CH, _CH)],
            isems.at[i],
        ).wait()
        pltpu.make_async_copy(
            buf.at[pl.ds(i * _CH, _CH)],
            out_ref.at[pl.ds(i * _CH, _CH)],
            osems.at[i],
        ).start()
    for i in range(_NC):
        pltpu.make_async_copy(
            buf.at[pl.ds(i * _CH, _CH)],
            out_ref.at[pl.ds(i * _CH, _CH)],
            osems.at[i],
        ).wait()


def kernel(x):
    B, T, D = x.shape
    xt = jax.lax.transpose(x, (1, 0, 2))
    yt = pl.pallas_call(
        _copy_body,
        out_shape=jax.ShapeDtypeStruct((T, B, D), x.dtype),
        in_specs=[pl.BlockSpec(memory_space=pl.ANY)],
        out_specs=pl.BlockSpec(memory_space=pl.ANY),
        scratch_shapes=[
            pltpu.VMEM((T, B, D), x.dtype),
            pltpu.SemaphoreType.DMA((_NC,)),
            pltpu.SemaphoreType.DMA((_NC,)),
        ],
    )(xt)
    return jax.lax.transpose(yt, (1, 0, 2))


# speedup vs baseline: 1.0093x; 1.0093x over previous
import jax
import jax.numpy as jnp
from jax.experimental import pallas as pl
from jax.experimental.pallas import tpu as pltpu

_NC = 2
_CH = 196 // _NC


def _copy_body(in_ref, out_ref, buf, isems, osems):
    for i in range(_NC):
        pltpu.make_async_copy(
            in_ref.at[pl.ds(i * _CH, _CH)],
            buf.at[pl.ds(i * _CH, _CH)],
            isems.at[i],
        ).start()
    for i in range(_NC):
        pltpu.make_async_copy(
            in_ref.at[pl.ds(i * _CH, _CH)],
            buf.at[pl.ds(i * _CH, _CH)],
            isems.at[i],
        ).wait()
        pltpu.make_async_copy(
            buf.at[pl.ds(i * _CH, _CH)],
            out_ref.at[pl.ds(i * _CH, _CH)],
            osems.at[i],
        ).start()
    for i in range(_NC):
        pltpu.make_async_copy(
            buf.at[pl.ds(i * _CH, _CH)],
            out_ref.at[pl.ds(i * _CH, _CH)],
            osems.at[i],
        ).wait()


def kernel(x):
    B, T, D = x.shape
    xt = jax.lax.transpose(x, (1, 0, 2))
    yt = pl.pallas_call(
        _copy_body,
        out_shape=jax.ShapeDtypeStruct((T, B, D), x.dtype),
        in_specs=[pl.BlockSpec(memory_space=pl.ANY)],
        out_specs=pl.BlockSpec(memory_space=pl.ANY),
        scratch_shapes=[
            pltpu.VMEM((T, B, D), x.dtype),
            pltpu.SemaphoreType.DMA((_NC,)),
            pltpu.SemaphoreType.DMA((_NC,)),
        ],
    )(xt)
    return jax.lax.transpose(yt, (1, 0, 2))
